# baseline (speedup 1.0000x reference)
import jax
import jax.numpy as jnp
from jax import lax
from jax.experimental import pallas as pl
from jax.experimental.pallas import tpu as pltpu
from jax.experimental.pallas import tpu_sc as plsc

N = 16384
D = 4096
L = 16
NUM_WORKERS = 32
ROWS_PER_TILE = N // NUM_WORKERS
R = 8
CHUNKS = ROWS_PER_TILE // R


def _body(x_hbm, perm_hbm, out_hbm, perm_v, in_v, out_v):
    num_cores = 2
    wid = lax.axis_index("s") * num_cores + lax.axis_index("c")
    row0 = wid * ROWS_PER_TILE

    pltpu.sync_copy(perm_hbm, perm_v)

    @pl.loop(0, CHUNKS)
    def _chunk(g):
        base = (row0 + g * R) * D
        pltpu.sync_copy(x_hbm.at[pl.ds(base, R * D)], in_v)

        @pl.loop(0, D // L)
        def _col(j):
            idx = perm_v[pl.ds(j * L, L)]
            for r in range(R):
                out_v[pl.ds(r * D + j * L, L)] = plsc.load_gather(
                    in_v, [idx + (r * D)])

        pltpu.sync_copy(out_v, out_hbm.at[pl.ds(base, R * D)])


def kernel(x, perm):
    mesh = plsc.VectorSubcoreMesh(core_axis_name="c", subcore_axis_name="s")
    f = pl.kernel(
        _body,
        out_type=jax.ShapeDtypeStruct((N * D,), jnp.float32),
        mesh=mesh,
        compiler_params=pltpu.CompilerParams(
            use_tc_tiling_on_sc=False, needs_layout_passes=False),
        scratch_types=[
            pltpu.VMEM((D,), jnp.int32),
            pltpu.VMEM((R * D,), jnp.float32),
            pltpu.VMEM((R * D,), jnp.float32),
        ],
    )
    return f(x.reshape(-1), perm.astype(jnp.int32)).reshape(N, D)


# speedup vs baseline: 1.8984x; 1.8984x over previous
import jax
import jax.numpy as jnp
from jax import lax
from jax.experimental import pallas as pl
from jax.experimental.pallas import tpu as pltpu
from jax.experimental.pallas import tpu_sc as plsc

N = 16384
D = 4096
L = 16
NUM_WORKERS = 32
ROWS_PER_TILE = N // NUM_WORKERS
R = 4
CB = R * D
CHUNKS = ROWS_PER_TILE // R
HALF = CHUNKS // 2


def _body(x_hbm, perm_hbm, out_hbm, perm_v,
          in0, in1, out0, out1, sin0, sin1, sout0, sout1):
    num_cores = 2
    wid = lax.axis_index("s") * num_cores + lax.axis_index("c")
    base0 = wid * ROWS_PER_TILE * D

    pltpu.sync_copy(perm_hbm, perm_v)

    def in_copy(g, buf, sem):
        return pltpu.make_async_copy(x_hbm.at[pl.ds(base0 + g * CB, CB)],
                                     buf, sem)

    def out_copy(g, buf, sem):
        return pltpu.make_async_copy(buf,
                                     out_hbm.at[pl.ds(base0 + g * CB, CB)],
                                     sem)

    def compute(in_b, out_b):
        @plsc.parallel_loop(0, D // L, unroll=8)
        def _col(j):
            idx = perm_v[pl.ds(j * L, L)]
            for r in range(R):
                out_b[pl.ds(r * D + j * L, L)] = plsc.load_gather(
                    in_b, [idx + (r * D)])

    slots = ((in0, out0, sin0, sout0), (in1, out1, sin1, sout1))

    in_copy(0, in0, sin0).start()
    in_copy(1, in1, sin1).start()

    @pl.loop(0, HALF)
    def _iter(i):
        for b, (inb, outb, sin, sout) in enumerate(slots):
            g = i * 2 + b
            in_copy(g, inb, sin).wait()

            @pl.when(i >= 1)
            def _():
                out_copy(g - 2, outb, sout).wait()

            compute(inb, outb)
            out_copy(g, outb, sout).start()

            @pl.when(i < HALF - 1)
            def _():
                in_copy(g + 2, inb, sin).start()

    out_copy(CHUNKS - 2, out0, sout0).wait()
    out_copy(CHUNKS - 1, out1, sout1).wait()


def kernel(x, perm):
    mesh = plsc.VectorSubcoreMesh(core_axis_name="c", subcore_axis_name="s")
    f = pl.kernel(
        _body,
        out_type=jax.ShapeDtypeStruct((N * D,), jnp.float32),
        mesh=mesh,
        compiler_params=pltpu.CompilerParams(
            use_tc_tiling_on_sc=False, needs_layout_passes=False),
        scratch_types=[
            pltpu.VMEM((D,), jnp.int32),
            pltpu.VMEM((CB,), jnp.float32),
            pltpu.VMEM((CB,), jnp.float32),
            pltpu.VMEM((CB,), jnp.float32),
            pltpu.VMEM((CB,), jnp.float32),
            pltpu.SemaphoreType.DMA,
            pltpu.SemaphoreType.DMA,
            pltpu.SemaphoreType.DMA,
            pltpu.SemaphoreType.DMA,
        ],
    )
    return f(x.reshape(-1), perm.astype(jnp.int32)).reshape(N, D)


# speedup vs baseline: 5.9279x; 3.1227x over previous
import jax
import jax.numpy as jnp
from jax import lax
from jax.experimental import pallas as pl
from jax.experimental.pallas import tpu as pltpu
from jax.experimental.pallas import tpu_sc as plsc

N = 16384
D = 4096
L = 16
NUM_WORKERS = 32
ROWS_PER_TILE = N // NUM_WORKERS
R = 4
CHUNKS = ROWS_PER_TILE // R
HALF = CHUNKS // 2


def _body(x_hbm, perm_hbm, out_hbm, perm_v,
          in0, in1, out0, out1, sin0, sin1, sout0, sout1):
    num_cores = 2
    wid = lax.axis_index("s") * num_cores + lax.axis_index("c")
    row0 = wid * ROWS_PER_TILE

    pltpu.sync_copy(perm_hbm, perm_v)

    def in_copy(g, buf, sem):
        return pltpu.make_async_copy(
            x_hbm.at[pl.ds(row0 + g * R, R), :], buf, sem)

    def out_copy(g, buf, sem):
        return pltpu.make_async_copy(
            buf, out_hbm.at[pl.ds(row0 + g * R, R), :], sem)

    def compute(in_b, out_b):
        @plsc.parallel_loop(0, D // L, unroll=8)
        def _col(j):
            idx = perm_v[pl.ds(j * L, L)]
            for r in range(R):
                row_ids = jnp.full((L,), r, jnp.int32)
                out_b[r, pl.ds(j * L, L)] = plsc.load_gather(
                    in_b, [row_ids, idx])

    slots = ((in0, out0, sin0, sout0), (in1, out1, sin1, sout1))

    in_copy(0, in0, sin0).start()
    in_copy(1, in1, sin1).start()

    @pl.loop(0, HALF)
    def _iter(i):
        for b, (inb, outb, sin, sout) in enumerate(slots):
            g = i * 2 + b
            in_copy(g, inb, sin).wait()

            @pl.when(i >= 1)
            def _():
                out_copy(g - 2, outb, sout).wait()

            compute(inb, outb)
            out_copy(g, outb, sout).start()

            @pl.when(i < HALF - 1)
            def _():
                in_copy(g + 2, inb, sin).start()

    out_copy(CHUNKS - 2, out0, sout0).wait()
    out_copy(CHUNKS - 1, out1, sout1).wait()


def kernel(x, perm):
    mesh = plsc.VectorSubcoreMesh(core_axis_name="c", subcore_axis_name="s")
    f = pl.kernel(
        _body,
        out_type=jax.ShapeDtypeStruct((N, D), jnp.float32),
        mesh=mesh,
        compiler_params=pltpu.CompilerParams(
            use_tc_tiling_on_sc=True, needs_layout_passes=False),
        scratch_types=[
            pltpu.VMEM((D,), jnp.int32),
            pltpu.VMEM((R, D), jnp.float32),
            pltpu.VMEM((R, D), jnp.float32),
            pltpu.VMEM((R, D), jnp.float32),
            pltpu.VMEM((R, D), jnp.float32),
            pltpu.SemaphoreType.DMA,
            pltpu.SemaphoreType.DMA,
            pltpu.SemaphoreType.DMA,
            pltpu.SemaphoreType.DMA,
        ],
    )
    return f(x, perm.astype(jnp.int32))


# speedup vs baseline: 6.0634x; 1.0228x over previous
import jax
import jax.numpy as jnp
from jax import lax
from jax.experimental import pallas as pl
from jax.experimental.pallas import tpu as pltpu
from jax.experimental.pallas import tpu_sc as plsc

N = 16384
D = 4096
L = 16
NUM_WORKERS = 32
ROWS_PER_TILE = N // NUM_WORKERS
R = 4
CHUNKS = ROWS_PER_TILE // R
NBUF_IN = 4
NBUF_OUT = 2
GROUPS = CHUNKS // NBUF_IN


def _body(x_hbm, perm_hbm, out_hbm, perm_v,
          in0, in1, in2, in3, out0, out1,
          sin0, sin1, sin2, sin3, sout0, sout1):
    num_cores = 2
    wid = lax.axis_index("s") * num_cores + lax.axis_index("c")
    row0 = wid * ROWS_PER_TILE

    pltpu.sync_copy(perm_hbm, perm_v)

    def in_copy(g, buf, sem):
        return pltpu.make_async_copy(
            x_hbm.at[pl.ds(row0 + g * R, R), :], buf, sem)

    def out_copy(g, buf, sem):
        return pltpu.make_async_copy(
            buf, out_hbm.at[pl.ds(row0 + g * R, R), :], sem)

    def compute(in_b, out_b):
        @plsc.parallel_loop(0, D // L, unroll=8)
        def _col(j):
            idx = perm_v[pl.ds(j * L, L)]
            for r in range(R):
                row_ids = jnp.full((L,), r, jnp.int32)
                out_b[r, pl.ds(j * L, L)] = plsc.load_gather(
                    in_b, [row_ids, idx])

    in_slots = ((in0, sin0), (in1, sin1), (in2, sin2), (in3, sin3))
    out_slots = ((out0, sout0), (out1, sout1))

    for b, (inb, sin) in enumerate(in_slots):
        in_copy(b, inb, sin).start()

    @pl.loop(0, GROUPS)
    def _iter(i):
        for b, (inb, sin) in enumerate(in_slots):
            outb, sout = out_slots[b % NBUF_OUT]
            g = i * NBUF_IN + b
            in_copy(g, inb, sin).wait()

            @pl.when(g >= NBUF_OUT)
            def _():
                out_copy(g - NBUF_OUT, outb, sout).wait()

            compute(inb, outb)
            out_copy(g, outb, sout).start()

            @pl.when(i < GROUPS - 1)
            def _():
                in_copy(g + NBUF_IN, inb, sin).start()

    out_copy(CHUNKS - 2, out0, sout0).wait()
    out_copy(CHUNKS - 1, out1, sout1).wait()


def kernel(x, perm):
    mesh = plsc.VectorSubcoreMesh(core_axis_name="c", subcore_axis_name="s")
    f = pl.kernel(
        _body,
        out_type=jax.ShapeDtypeStruct((N, D), jnp.float32),
        mesh=mesh,
        compiler_params=pltpu.CompilerParams(
            use_tc_tiling_on_sc=True, needs_layout_passes=False),
        scratch_types=[
            pltpu.VMEM((D,), jnp.int32),
            pltpu.VMEM((R, D), jnp.float32),
            pltpu.VMEM((R, D), jnp.float32),
            pltpu.VMEM((R, D), jnp.float32),
            pltpu.VMEM((R, D), jnp.float32),
            pltpu.VMEM((R, D), jnp.float32),
            pltpu.VMEM((R, D), jnp.float32),
            pltpu.SemaphoreType.DMA,
            pltpu.SemaphoreType.DMA,
            pltpu.SemaphoreType.DMA,
            pltpu.SemaphoreType.DMA,
            pltpu.SemaphoreType.DMA,
            pltpu.SemaphoreType.DMA,
        ],
    )
    return f(x, perm.astype(jnp.int32))
